# baseline (speedup 1.0000x reference)
import jax
import jax.numpy as jnp
from jax.experimental import pallas as pl
from jax.experimental.pallas import tpu as pltpu

N = 10000
N_FEAT = 128
N_HID = 128
N_CLASS = 40
BLK1 = 200
N_TILES1 = N // BLK1
BLK2 = 400
N_TILES2 = N // BLK2


def _pass1_kernel(x_ref, gso_ref, w1_ref, b1_ref, w2_ref,
                  gso16_ref, s2_ref, support_ref):
    i = pl.program_id(0)

    @pl.when(i == 0)
    def _():
        support_ref[...] = jnp.dot(
            x_ref[...], w1_ref[...],
            preferred_element_type=jnp.float32).astype(jnp.bfloat16)

    g16 = gso_ref[...].astype(jnp.bfloat16)
    gso16_ref[...] = g16
    acc = jnp.dot(g16, support_ref[...], preferred_element_type=jnp.float32)
    h = jnp.maximum(acc + b1_ref[...], 0.0)
    s2_ref[...] = jnp.dot(
        h, w2_ref[...], preferred_element_type=jnp.float32).astype(jnp.bfloat16)


def _pass2_kernel(gso16_ref, s2_ref, b2_ref, wlin_ref, blin_ref, o_ref):
    acc = jnp.dot(gso16_ref[...], s2_ref[...],
                  preferred_element_type=jnp.float32)
    h = jnp.maximum(acc + b2_ref[...], 0.0)
    logits = jnp.dot(h, wlin_ref[...],
                     preferred_element_type=jnp.float32) + blin_ref[...]
    m = jnp.max(logits, axis=1, keepdims=True)
    shifted = logits - m
    lse = jnp.log(jnp.sum(jnp.exp(shifted), axis=1, keepdims=True))
    o_ref[...] = shifted - lse


def kernel(x, gso_real, gso_imag, W1, b1, W2, b2, Wlin, blin):
    del gso_imag
    b1r = b1.reshape(1, N_HID)
    b2r = b2.reshape(1, N_HID)
    blinr = blin.reshape(1, N_CLASS)

    gso16, s2 = pl.pallas_call(
        _pass1_kernel,
        grid=(N_TILES1,),
        out_shape=(
            jax.ShapeDtypeStruct((N, N), jnp.bfloat16),
            jax.ShapeDtypeStruct((N, N_HID), jnp.bfloat16),
        ),
        in_specs=[
            pl.BlockSpec((N, N_FEAT), lambda i: (0, 0)),
            pl.BlockSpec((BLK1, N), lambda i: (i, 0)),
            pl.BlockSpec((N_FEAT, N_HID), lambda i: (0, 0)),
            pl.BlockSpec((1, N_HID), lambda i: (0, 0)),
            pl.BlockSpec((N_HID, N_HID), lambda i: (0, 0)),
        ],
        out_specs=(
            pl.BlockSpec((BLK1, N), lambda i: (i, 0)),
            pl.BlockSpec((BLK1, N_HID), lambda i: (i, 0)),
        ),
        scratch_shapes=[pltpu.VMEM((N, N_HID), jnp.bfloat16)],
        compiler_params=pltpu.CompilerParams(
            dimension_semantics=("arbitrary",)),
    )(x, gso_real, W1, b1r, W2)

    out = pl.pallas_call(
        _pass2_kernel,
        grid=(N_TILES2,),
        out_shape=jax.ShapeDtypeStruct((N, N_CLASS), jnp.float32),
        in_specs=[
            pl.BlockSpec((BLK2, N), lambda i: (i, 0)),
            pl.BlockSpec((N, N_HID), lambda i: (0, 0)),
            pl.BlockSpec((1, N_HID), lambda i: (0, 0)),
            pl.BlockSpec((N_HID, N_CLASS), lambda i: (0, 0)),
            pl.BlockSpec((1, N_CLASS), lambda i: (0, 0)),
        ],
        out_specs=pl.BlockSpec((BLK2, N_CLASS), lambda i: (i, 0)),
        compiler_params=pltpu.CompilerParams(
            dimension_semantics=("arbitrary",)),
    )(gso16, s2, b2r, Wlin, blinr)

    return out


# speedup vs baseline: 1.1039x; 1.1039x over previous
import jax
import jax.numpy as jnp
from jax.experimental import pallas as pl
from jax.experimental.pallas import tpu as pltpu

N = 10000
N_FEAT = 128
N_HID = 128
N_CLASS = 40
ROW_BLK = 400
N_TILES = N // ROW_BLK


def _fused_kernel(x_ref, gso_ref, w1_ref, b1_ref, w2_ref, b2_ref,
                  wlin_ref, blin_ref, o_ref, support_ref, s2_ref):
    p = pl.program_id(0)
    i = pl.program_id(1)

    @pl.when(jnp.logical_and(p == 0, i == 0))
    def _():
        support_ref[...] = jnp.dot(x_ref[...], w1_ref[...],
                                   preferred_element_type=jnp.float32)

    @pl.when(p == 0)
    def _():
        acc = jnp.dot(gso_ref[...], support_ref[...],
                      preferred_element_type=jnp.float32)
        h = jnp.maximum(acc + b1_ref[...], 0.0)
        s2_ref[pl.ds(i * ROW_BLK, ROW_BLK), :] = jnp.dot(
            h, w2_ref[...], preferred_element_type=jnp.float32)

    @pl.when(p == 1)
    def _():
        acc = jnp.dot(gso_ref[...], s2_ref[...],
                      preferred_element_type=jnp.float32)
        h = jnp.maximum(acc + b2_ref[...], 0.0)
        logits = jnp.dot(h, wlin_ref[...],
                         preferred_element_type=jnp.float32) + blin_ref[...]
        m = jnp.max(logits, axis=1, keepdims=True)
        shifted = logits - m
        lse = jnp.log(jnp.sum(jnp.exp(shifted), axis=1, keepdims=True))
        o_ref[...] = shifted - lse


def kernel(x, gso_real, gso_imag, W1, b1, W2, b2, Wlin, blin):
    del gso_imag
    b1r = b1.reshape(1, N_HID)
    b2r = b2.reshape(1, N_HID)
    blinr = blin.reshape(1, N_CLASS)

    out = pl.pallas_call(
        _fused_kernel,
        grid=(2, N_TILES),
        out_shape=jax.ShapeDtypeStruct((N, N_CLASS), jnp.float32),
        in_specs=[
            pl.BlockSpec((N, N_FEAT), lambda p, i: (0, 0)),
            pl.BlockSpec((ROW_BLK, N), lambda p, i: (i, 0)),
            pl.BlockSpec((N_FEAT, N_HID), lambda p, i: (0, 0)),
            pl.BlockSpec((1, N_HID), lambda p, i: (0, 0)),
            pl.BlockSpec((N_HID, N_HID), lambda p, i: (0, 0)),
            pl.BlockSpec((1, N_HID), lambda p, i: (0, 0)),
            pl.BlockSpec((N_HID, N_CLASS), lambda p, i: (0, 0)),
            pl.BlockSpec((1, N_CLASS), lambda p, i: (0, 0)),
        ],
        out_specs=pl.BlockSpec((ROW_BLK, N_CLASS), lambda p, i: (i, 0)),
        scratch_shapes=[
            pltpu.VMEM((N, N_HID), jnp.float32),
            pltpu.VMEM((N, N_HID), jnp.float32),
        ],
        compiler_params=pltpu.CompilerParams(
            dimension_semantics=("arbitrary", "arbitrary")),
    )(x, gso_real, W1, b1r, W2, b2r, Wlin, blinr)

    return out


# speedup vs baseline: 1.1041x; 1.0001x over previous
import jax
import jax.numpy as jnp
from jax.experimental import pallas as pl
from jax.experimental.pallas import tpu as pltpu

N = 10000
N_FEAT = 128
N_HID = 128
N_CLASS = 40
ROW_BLK = 400
N_TILES = N // ROW_BLK


def _fused_kernel(x_ref, gso_ref, w1_ref, b1_ref, w2_ref, b2_ref,
                  wlin_ref, blin_ref, o_ref, support_ref, s2_ref):
    p = pl.program_id(0)
    i = pl.program_id(1)

    @pl.when(jnp.logical_and(p == 0, i == 0))
    def _():
        support_ref[...] = jnp.dot(x_ref[...], w1_ref[...],
                                   preferred_element_type=jnp.float32)

    @pl.when(p == 0)
    def _():
        acc = jnp.dot(gso_ref[...], support_ref[...],
                      preferred_element_type=jnp.float32)
        h = jnp.maximum(acc + b1_ref[...], 0.0)
        s2_ref[pl.ds(i * ROW_BLK, ROW_BLK), :] = jnp.dot(
            h, w2_ref[...], preferred_element_type=jnp.float32)

    @pl.when(p == 1)
    def _():
        acc = jnp.dot(gso_ref[...], s2_ref[...],
                      preferred_element_type=jnp.float32)
        h = jnp.maximum(acc + b2_ref[...], 0.0)
        logits = jnp.dot(h, wlin_ref[...],
                         preferred_element_type=jnp.float32) + blin_ref[...]
        m = jnp.max(logits, axis=1, keepdims=True)
        shifted = logits - m
        lse = jnp.log(jnp.sum(jnp.exp(shifted), axis=1, keepdims=True))
        o_ref[...] = shifted - lse


def kernel(x, gso_real, gso_imag, W1, b1, W2, b2, Wlin, blin):
    del gso_imag
    b1r = b1.reshape(1, N_HID)
    b2r = b2.reshape(1, N_HID)
    blinr = blin.reshape(1, N_CLASS)

    out = pl.pallas_call(
        _fused_kernel,
        grid=(2, N_TILES),
        out_shape=jax.ShapeDtypeStruct((N, N_CLASS), jnp.float32),
        in_specs=[
            pl.BlockSpec((N, N_FEAT), lambda p, i: (0, 0)),
            pl.BlockSpec((ROW_BLK, N), lambda p, i: (i, 0)),
            pl.BlockSpec((N_FEAT, N_HID), lambda p, i: (0, 0)),
            pl.BlockSpec((1, N_HID), lambda p, i: (0, 0)),
            pl.BlockSpec((N_HID, N_HID), lambda p, i: (0, 0)),
            pl.BlockSpec((1, N_HID), lambda p, i: (0, 0)),
            pl.BlockSpec((N_HID, N_CLASS), lambda p, i: (0, 0)),
            pl.BlockSpec((1, N_CLASS), lambda p, i: (0, 0)),
        ],
        out_specs=pl.BlockSpec((ROW_BLK, N_CLASS), lambda p, i: (i, 0)),
        scratch_shapes=[
            pltpu.VMEM((N, N_HID), jnp.float32),
            pltpu.VMEM((N, N_HID), jnp.float32),
        ],
        compiler_params=pltpu.CompilerParams(
            dimension_semantics=("arbitrary", "parallel")),
    )(x, gso_real, W1, b1r, W2, b2r, Wlin, blinr)

    return out
